# baseline (speedup 1.0000x reference)
import jax
import jax.numpy as jnp
from jax import lax
from jax.experimental import pallas as pl
from jax.experimental.pallas import tpu as pltpu
from jax.experimental.pallas import tpu_sc as plsc

N_NODES = 10000
N_EDGES = 320000
F_DIM = 128
S_DIM = 16
N_ROUNDS = 4
N_GRAPHS = 64

NC = 2
NS = 16

CHUNK = 128
K_GRP = 8
CH_PER_TILE = 160
N_CHUNKS = NS * CH_PER_TILE
E_PAD = N_CHUNKS * CHUNK
GRPS_PER_TILE = CH_PER_TILE // K_GRP
N_PAD = 10240
ROWS_PER_TILE = N_PAD // NS
RBLK = 128
N_RBLK = ROWS_PER_TILE // RBLK
GS_ROWS = 128

BLK = 1024
N_BLKS = N_PAD // BLK


def _sc_body(st0_hbm, msg0_hbm, edge_hbm, batch_hbm, wm_hbm, bm_hbm, wu_hbm,
             bu_hbm, wo_hbm, bo_hbm, out_hbm, msgscr_hbm,
             src_v, dst_v, gbuf, zbuf, abuf, sbuf, s2buf, mbuf, bidx_v,
             wm_v, bm_v, wu_v, bu_v, wo_v, bo_v, gsbuf, obuf,
             state_sh, agg, gs,
             sem_i, sem_ga, sem_gb, sem_sa, sem_sb):
    c = lax.axis_index("c")
    s = lax.axis_index("s")
    K = K_GRP
    base = s * CH_PER_TILE
    rows_t = pl.ds(s * ROWS_PER_TILE, ROWS_PER_TILE)

    pltpu.async_copy(edge_hbm.at[0, pl.ds(base, CH_PER_TILE)], src_v, sem_i)
    pltpu.async_copy(edge_hbm.at[1, pl.ds(base, CH_PER_TILE)], dst_v, sem_i)
    pltpu.sync_copy(st0_hbm.at[rows_t], state_sh.at[rows_t])
    pltpu.sync_copy(wm_hbm, wm_v)
    pltpu.sync_copy(bm_hbm, bm_v)
    pltpu.sync_copy(wu_hbm, wu_v)
    pltpu.sync_copy(bu_hbm, bu_v)
    pltpu.sync_copy(wo_hbm, wo_v)
    pltpu.sync_copy(bo_hbm, bo_v)
    pltpu.sync_copy(batch_hbm.at[pl.ds(N_RBLK * s, N_RBLK)], bidx_v)

    zrow = jnp.zeros((S_DIM,), jnp.float32)

    def _zb(i, _):
        zbuf[i] = zrow
        return 0

    lax.fori_loop(0, ROWS_PER_TILE, _zb, 0)
    pltpu.sync_copy(zbuf, agg.at[rows_t])

    @pl.when(s == 0)
    def _():
        pltpu.sync_copy(zbuf.at[pl.ds(0, GS_ROWS)], gs)

    pltpu.make_async_copy(edge_hbm.at[0, pl.ds(base, CH_PER_TILE)], src_v,
                          sem_i).wait()
    pltpu.make_async_copy(edge_hbm.at[1, pl.ds(base, CH_PER_TILE)], dst_v,
                          sem_i).wait()
    plsc.subcore_barrier()

    sem_g = (sem_ga, sem_gb)
    sem_s = (sem_sa, sem_sb)

    def edge_phase(msg_ref):
        def fire_gathers(g, h):
            for b in range(K):
                pltpu.async_copy(msg_ref.at[src_v.at[g * K + b]],
                                 gbuf.at[h * K + b], sem_g[h])

        def wait_gathers(g, h):
            for b in range(K):
                pltpu.make_async_copy(msg_ref.at[src_v.at[g * K + b]],
                                      gbuf.at[h * K + b], sem_g[h]).wait()

        def fire_scatters(g, h):
            for b in range(K):
                pltpu.async_copy(gbuf.at[h * K + b], agg.at[dst_v.at[g * K + b]],
                                 sem_s[h], add=True)

        def drain_scatters(g, h):
            for b in range(K):
                pltpu.make_async_copy(gbuf.at[h * K + b],
                                      agg.at[dst_v.at[g * K + b]],
                                      sem_s[h]).wait()

        fire_gathers(0, 0)

        def _outer(i, _):
            q = i * 2
            wait_gathers(q, 0)
            fire_scatters(q, 0)

            @pl.when(q > 0)
            def _():
                drain_scatters(q - 1, 1)

            fire_gathers(q + 1, 1)
            wait_gathers(q + 1, 1)
            fire_scatters(q + 1, 1)
            drain_scatters(q, 0)

            @pl.when(q < GRPS_PER_TILE - 2)
            def _():
                fire_gathers(q + 2, 0)

            return 0

        lax.fori_loop(0, GRPS_PER_TILE // 2, _outer, 0)
        drain_scatters(GRPS_PER_TILE - 1, 1)

    def update_phase(r):
        wu_rows = tuple(wu_v[r, k] for k in range(S_DIM))
        bu_row = bu_v[r]
        if r < N_ROUNDS - 1:
            wm_rows = tuple(wm_v[r + 1, k] for k in range(S_DIM))
            bm_row = bm_v[r + 1]
        else:
            wm_rows = ()
            bm_row = bu_row

        def _blk(blk, _):
            rows = pl.ds(s * ROWS_PER_TILE + blk * RBLK, RBLK)
            pltpu.sync_copy(agg.at[rows], abuf)
            pltpu.sync_copy(state_sh.at[rows], sbuf)
            pltpu.sync_copy(zbuf.at[pl.ds(0, RBLK)], agg.at[rows])

            def _node(i, w):
                wu_c, bu_c, wm_c, bm_c = w
                av = abuf[i]
                acc = bu_c
                for k in range(S_DIM):
                    acc = acc + wu_c[k] * av[k]
                nst = sbuf[i] + jnp.maximum(acc, 0.0)
                s2buf[i] = nst
                if r < N_ROUNDS - 1:
                    macc = bm_c
                    for k in range(S_DIM):
                        macc = macc + wm_c[k] * nst[k]
                    mbuf[i] = jnp.maximum(macc, 0.0)
                return w

            lax.fori_loop(0, RBLK, _node, (wu_rows, bu_row, wm_rows, bm_row))
            pltpu.sync_copy(s2buf, state_sh.at[rows])
            if r < N_ROUNDS - 1:
                pltpu.sync_copy(mbuf, msgscr_hbm.at[c, rows])
            else:
                pltpu.sync_copy(s2buf, gs.at[bidx_v.at[blk]], add=True)
            return 0

        lax.fori_loop(0, N_RBLK, _blk, 0)

    for r in range(N_ROUNDS):
        edge_phase(msg0_hbm if r == 0 else msgscr_hbm.at[c])
        plsc.subcore_barrier()
        update_phase(r)
        plsc.subcore_barrier()

    @pl.when((c == 0) & (s == 0))
    def _():
        pltpu.sync_copy(gs.at[pl.ds(0, N_GRAPHS)], gsbuf)
        wo_row = wo_v[0]
        bo_s = bo_v[0][0]
        lanes = lax.iota(jnp.int32, 16)

        def _g(rr, _):
            acc = jnp.zeros((16,), jnp.float32)
            for j in range(16):
                val = jnp.sum(gsbuf[rr * 16 + j] * wo_row) + bo_s
                acc = acc + jnp.where(lanes == j, val, 0.0)
            obuf[rr] = acc
            return 0

        lax.fori_loop(0, N_GRAPHS // 16, _g, 0)
        pltpu.sync_copy(obuf, out_hbm)


_sc_mega = pl.kernel(
    _sc_body,
    out_type=(
        jax.ShapeDtypeStruct((N_GRAPHS // 16, 16), jnp.float32),
        jax.ShapeDtypeStruct((NC, N_PAD, S_DIM), jnp.float32),
    ),
    mesh=plsc.VectorSubcoreMesh(core_axis_name="c", subcore_axis_name="s"),
    compiler_params=pltpu.CompilerParams(
        use_tc_tiling_on_sc=False, needs_layout_passes=False
    ),
    scratch_types=[
        pltpu.VMEM((CH_PER_TILE, CHUNK), jnp.int32),
        pltpu.VMEM((CH_PER_TILE, CHUNK), jnp.int32),
        pltpu.VMEM((2 * K_GRP, CHUNK, S_DIM), jnp.float32),
        pltpu.VMEM((ROWS_PER_TILE, S_DIM), jnp.float32),
        pltpu.VMEM((RBLK, S_DIM), jnp.float32),
        pltpu.VMEM((RBLK, S_DIM), jnp.float32),
        pltpu.VMEM((RBLK, S_DIM), jnp.float32),
        pltpu.VMEM((RBLK, S_DIM), jnp.float32),
        pltpu.VMEM((N_RBLK, CHUNK), jnp.int32),
        pltpu.VMEM((N_ROUNDS, S_DIM, S_DIM), jnp.float32),
        pltpu.VMEM((N_ROUNDS, S_DIM), jnp.float32),
        pltpu.VMEM((N_ROUNDS, S_DIM, S_DIM), jnp.float32),
        pltpu.VMEM((N_ROUNDS, S_DIM), jnp.float32),
        pltpu.VMEM((1, S_DIM), jnp.float32),
        pltpu.VMEM((1, S_DIM), jnp.float32),
        pltpu.VMEM((N_GRAPHS, S_DIM), jnp.float32),
        pltpu.VMEM((N_GRAPHS // 16, 16), jnp.float32),
        pltpu.VMEM_SHARED((N_PAD, S_DIM), jnp.float32),
        pltpu.VMEM_SHARED((N_PAD, S_DIM), jnp.float32),
        pltpu.VMEM_SHARED((GS_ROWS, S_DIM), jnp.float32),
        pltpu.SemaphoreType.DMA,
        pltpu.SemaphoreType.DMA,
        pltpu.SemaphoreType.DMA,
        pltpu.SemaphoreType.DMA,
        pltpu.SemaphoreType.DMA,
    ],
)


def _tc_init_body(x_ref, wi_ref, bi_ref, wm_ref, bm_ref, st_ref, msg_ref):
    st = jnp.maximum(
        jnp.dot(x_ref[...], wi_ref[...], preferred_element_type=jnp.float32)
        + bi_ref[...],
        0.0,
    )
    st_ref[...] = st
    msg_ref[...] = jnp.maximum(
        jnp.dot(st, wm_ref[...], preferred_element_type=jnp.float32) + bm_ref[...],
        0.0,
    )


def _tc_init(x, wi, bi, wm, bm):
    return pl.pallas_call(
        _tc_init_body,
        grid=(N_BLKS,),
        in_specs=[
            pl.BlockSpec((BLK, F_DIM), lambda i: (i, 0)),
            pl.BlockSpec((F_DIM, S_DIM), lambda i: (0, 0)),
            pl.BlockSpec((1, S_DIM), lambda i: (0, 0)),
            pl.BlockSpec((S_DIM, S_DIM), lambda i: (0, 0)),
            pl.BlockSpec((1, S_DIM), lambda i: (0, 0)),
        ],
        out_specs=[
            pl.BlockSpec((BLK, S_DIM), lambda i: (i, 0)),
            pl.BlockSpec((BLK, S_DIM), lambda i: (i, 0)),
        ],
        out_shape=[
            jax.ShapeDtypeStruct((N_PAD, S_DIM), jnp.float32),
            jax.ShapeDtypeStruct((N_PAD, S_DIM), jnp.float32),
        ],
    )(x, wi, bi, wm, bm)


def kernel(x, edge_index, batch, Wi, bi, Wm, bm, Wu, bu, Wo, bo):
    pad = jnp.concatenate(
        [
            jnp.zeros((1, E_PAD - N_EDGES), jnp.int32),
            jnp.full((1, E_PAD - N_EDGES), N_NODES, jnp.int32),
        ],
        axis=0,
    )
    edge3 = jnp.concatenate([edge_index, pad], axis=1).reshape(2, N_CHUNKS, CHUNK)
    batch2 = jnp.concatenate(
        [batch, jnp.full((N_PAD - N_NODES,), N_GRAPHS, jnp.int32)]
    ).reshape(N_PAD // CHUNK, CHUNK)

    x_pad = jnp.concatenate(
        [x, jnp.zeros((N_PAD - N_NODES, F_DIM), jnp.float32)], axis=0
    )
    st0, msg0 = _tc_init(
        x_pad, Wi, bi.reshape(1, S_DIM), Wm[0], bm[0].reshape(1, S_DIM)
    )
    out, _ = _sc_mega(
        st0, msg0, edge3, batch2,
        Wm, bm, Wu, bu,
        Wo.reshape(1, S_DIM), jnp.broadcast_to(bo.reshape(1, 1), (1, S_DIM)),
    )
    return out.reshape(-1)


# speedup vs baseline: 1.1966x; 1.1966x over previous
import jax
import jax.numpy as jnp
from jax import lax
from jax.experimental import pallas as pl
from jax.experimental.pallas import tpu as pltpu
from jax.experimental.pallas import tpu_sc as plsc

N_NODES = 10000
N_EDGES = 320000
F_DIM = 128
S_DIM = 16
N_ROUNDS = 4
N_GRAPHS = 64

NC = 2
NS = 16

CHUNK = 128
K_GRP = 8
CH_PER_TILE = 80
N_CHUNKS = NC * NS * CH_PER_TILE
E_PAD = N_CHUNKS * CHUNK
GRPS_PER_TILE = CH_PER_TILE // K_GRP
N_PAD = 10240
ROWS_PER_TILE = N_PAD // NS
RBLK = 128
N_RBLK = ROWS_PER_TILE // RBLK
GS_ROWS = 128

BLK = 1024
N_BLKS = N_PAD // BLK


def _sc_body(st0_hbm, msg0_hbm, edge_hbm, batch_hbm, wm_hbm, bm_hbm, wu_hbm,
             bu_hbm, wo_hbm, bo_hbm, out_hbm, msgscr_hbm, pout_hbm,
             src_v, dst_v, gbuf, zbuf, abuf, pbuf, sbuf, s2buf, mbuf, bidx_v,
             wm_v, bm_v, wu_v, bu_v, wo_v, bo_v, gsbuf, obuf,
             state_sh, agg, gs,
             sem_i, sem_ga, sem_gb, sem_sa, sem_sb, xsem):
    c = lax.axis_index("c")
    s = lax.axis_index("s")
    K = K_GRP
    base = (c * NS + s) * CH_PER_TILE
    rows_t = pl.ds(s * ROWS_PER_TILE, ROWS_PER_TILE)

    pltpu.async_copy(edge_hbm.at[0, pl.ds(base, CH_PER_TILE)], src_v, sem_i)
    pltpu.async_copy(edge_hbm.at[1, pl.ds(base, CH_PER_TILE)], dst_v, sem_i)
    pltpu.sync_copy(st0_hbm.at[rows_t], state_sh.at[rows_t])
    pltpu.sync_copy(wm_hbm, wm_v)
    pltpu.sync_copy(bm_hbm, bm_v)
    pltpu.sync_copy(wu_hbm, wu_v)
    pltpu.sync_copy(bu_hbm, bu_v)
    pltpu.sync_copy(wo_hbm, wo_v)
    pltpu.sync_copy(bo_hbm, bo_v)
    pltpu.sync_copy(batch_hbm.at[pl.ds(N_RBLK * s, N_RBLK)], bidx_v)

    zrow = jnp.zeros((S_DIM,), jnp.float32)

    def _zb(i, _):
        zbuf[i] = zrow
        return 0

    lax.fori_loop(0, ROWS_PER_TILE, _zb, 0)
    pltpu.sync_copy(zbuf, agg.at[rows_t])

    @pl.when(s == 0)
    def _():
        pltpu.sync_copy(zbuf.at[pl.ds(0, GS_ROWS)], gs)

    pltpu.make_async_copy(edge_hbm.at[0, pl.ds(base, CH_PER_TILE)], src_v,
                          sem_i).wait()
    pltpu.make_async_copy(edge_hbm.at[1, pl.ds(base, CH_PER_TILE)], dst_v,
                          sem_i).wait()
    plsc.subcore_barrier()

    sem_g = (sem_ga, sem_gb)
    sem_s = (sem_sa, sem_sb)

    def edge_phase(msg_ref):
        def fire_gathers(g, h):
            for b in range(K):
                pltpu.async_copy(msg_ref.at[src_v.at[g * K + b]],
                                 gbuf.at[h * K + b], sem_g[h])

        def wait_gathers(g, h):
            for b in range(K):
                pltpu.make_async_copy(msg_ref.at[src_v.at[g * K + b]],
                                      gbuf.at[h * K + b], sem_g[h]).wait()

        def fire_scatters(g, h):
            for b in range(K):
                pltpu.async_copy(gbuf.at[h * K + b], agg.at[dst_v.at[g * K + b]],
                                 sem_s[h], add=True)

        def drain_scatters(g, h):
            for b in range(K):
                pltpu.make_async_copy(gbuf.at[h * K + b],
                                      agg.at[dst_v.at[g * K + b]],
                                      sem_s[h]).wait()

        fire_gathers(0, 0)

        def _outer(i, _):
            q = i * 2
            wait_gathers(q, 0)
            fire_scatters(q, 0)

            @pl.when(q > 0)
            def _():
                drain_scatters(q - 1, 1)

            fire_gathers(q + 1, 1)
            wait_gathers(q + 1, 1)
            fire_scatters(q + 1, 1)
            drain_scatters(q, 0)

            @pl.when(q < GRPS_PER_TILE - 2)
            def _():
                fire_gathers(q + 2, 0)

            return 0

        lax.fori_loop(0, GRPS_PER_TILE // 2, _outer, 0)
        drain_scatters(GRPS_PER_TILE - 1, 1)

    def update_phase(r):
        wu_rows = tuple(wu_v[r, k] for k in range(S_DIM))
        bu_row = bu_v[r]
        if r < N_ROUNDS - 1:
            wm_rows = tuple(wm_v[r + 1, k] for k in range(S_DIM))
            bm_row = bm_v[r + 1]
        else:
            wm_rows = ()
            bm_row = bu_row

        def _blk(blk, _):
            rows = pl.ds(s * ROWS_PER_TILE + blk * RBLK, RBLK)
            pltpu.sync_copy(agg.at[rows], abuf)
            pltpu.sync_copy(pout_hbm.at[1 - c, rows], pbuf)
            pltpu.sync_copy(state_sh.at[rows], sbuf)
            pltpu.sync_copy(zbuf.at[pl.ds(0, RBLK)], agg.at[rows])

            def _node(i, w):
                wu_c, bu_c, wm_c, bm_c = w
                av = abuf[i] + pbuf[i]
                acc = bu_c
                for k in range(S_DIM):
                    acc = acc + wu_c[k] * av[k]
                nst = sbuf[i] + jnp.maximum(acc, 0.0)
                s2buf[i] = nst
                if r < N_ROUNDS - 1:
                    macc = bm_c
                    for k in range(S_DIM):
                        macc = macc + wm_c[k] * nst[k]
                    mbuf[i] = jnp.maximum(macc, 0.0)
                return w

            lax.fori_loop(0, RBLK, _node, (wu_rows, bu_row, wm_rows, bm_row))
            pltpu.sync_copy(s2buf, state_sh.at[rows])
            if r < N_ROUNDS - 1:
                pltpu.sync_copy(mbuf, msgscr_hbm.at[c, rows])
            else:
                pltpu.sync_copy(s2buf, gs.at[bidx_v.at[blk]], add=True)
            return 0

        lax.fori_loop(0, N_RBLK, _blk, 0)

    for r in range(N_ROUNDS):
        edge_phase(msg0_hbm if r == 0 else msgscr_hbm.at[c])
        plsc.subcore_barrier()
        pltpu.sync_copy(agg.at[rows_t], pout_hbm.at[c, rows_t])
        plsc.subcore_barrier()

        @pl.when(s == 0)
        def _():
            pl.semaphore_signal(xsem, 1, core_index=1 - c)
            pl.semaphore_wait(xsem, 1)

        plsc.subcore_barrier()
        update_phase(r)
        plsc.subcore_barrier()

    @pl.when((c == 0) & (s == 0))
    def _():
        pltpu.sync_copy(gs.at[pl.ds(0, N_GRAPHS)], gsbuf)
        wo_row = wo_v[0]
        bo_s = bo_v[0][0]
        lanes = lax.iota(jnp.int32, 16)

        def _g(rr, _):
            acc = jnp.zeros((16,), jnp.float32)
            for j in range(16):
                val = jnp.sum(gsbuf[rr * 16 + j] * wo_row) + bo_s
                acc = acc + jnp.where(lanes == j, val, 0.0)
            obuf[rr] = acc
            return 0

        lax.fori_loop(0, N_GRAPHS // 16, _g, 0)
        pltpu.sync_copy(obuf, out_hbm)


_sc_mega = pl.kernel(
    _sc_body,
    out_type=(
        jax.ShapeDtypeStruct((N_GRAPHS // 16, 16), jnp.float32),
        jax.ShapeDtypeStruct((NC, N_PAD, S_DIM), jnp.float32),
        jax.ShapeDtypeStruct((NC, N_PAD, S_DIM), jnp.float32),
    ),
    mesh=plsc.VectorSubcoreMesh(core_axis_name="c", subcore_axis_name="s"),
    compiler_params=pltpu.CompilerParams(
        use_tc_tiling_on_sc=False, needs_layout_passes=False
    ),
    scratch_types=[
        pltpu.VMEM((CH_PER_TILE, CHUNK), jnp.int32),
        pltpu.VMEM((CH_PER_TILE, CHUNK), jnp.int32),
        pltpu.VMEM((2 * K_GRP, CHUNK, S_DIM), jnp.float32),
        pltpu.VMEM((ROWS_PER_TILE, S_DIM), jnp.float32),
        pltpu.VMEM((RBLK, S_DIM), jnp.float32),
        pltpu.VMEM((RBLK, S_DIM), jnp.float32),
        pltpu.VMEM((RBLK, S_DIM), jnp.float32),
        pltpu.VMEM((RBLK, S_DIM), jnp.float32),
        pltpu.VMEM((RBLK, S_DIM), jnp.float32),
        pltpu.VMEM((N_RBLK, CHUNK), jnp.int32),
        pltpu.VMEM((N_ROUNDS, S_DIM, S_DIM), jnp.float32),
        pltpu.VMEM((N_ROUNDS, S_DIM), jnp.float32),
        pltpu.VMEM((N_ROUNDS, S_DIM, S_DIM), jnp.float32),
        pltpu.VMEM((N_ROUNDS, S_DIM), jnp.float32),
        pltpu.VMEM((1, S_DIM), jnp.float32),
        pltpu.VMEM((1, S_DIM), jnp.float32),
        pltpu.VMEM((N_GRAPHS, S_DIM), jnp.float32),
        pltpu.VMEM((N_GRAPHS // 16, 16), jnp.float32),
        pltpu.VMEM_SHARED((N_PAD, S_DIM), jnp.float32),
        pltpu.VMEM_SHARED((N_PAD, S_DIM), jnp.float32),
        pltpu.VMEM_SHARED((GS_ROWS, S_DIM), jnp.float32),
        pltpu.SemaphoreType.DMA,
        pltpu.SemaphoreType.DMA,
        pltpu.SemaphoreType.DMA,
        pltpu.SemaphoreType.DMA,
        pltpu.SemaphoreType.DMA,
        pltpu.SemaphoreType.REGULAR,
    ],
)


def _tc_init_body(x_ref, wi_ref, bi_ref, wm_ref, bm_ref, st_ref, msg_ref):
    st = jnp.maximum(
        jnp.dot(x_ref[...], wi_ref[...], preferred_element_type=jnp.float32)
        + bi_ref[...],
        0.0,
    )
    st_ref[...] = st
    msg_ref[...] = jnp.maximum(
        jnp.dot(st, wm_ref[...], preferred_element_type=jnp.float32) + bm_ref[...],
        0.0,
    )


def _tc_init(x, wi, bi, wm, bm):
    return pl.pallas_call(
        _tc_init_body,
        grid=(N_BLKS,),
        in_specs=[
            pl.BlockSpec((BLK, F_DIM), lambda i: (i, 0)),
            pl.BlockSpec((F_DIM, S_DIM), lambda i: (0, 0)),
            pl.BlockSpec((1, S_DIM), lambda i: (0, 0)),
            pl.BlockSpec((S_DIM, S_DIM), lambda i: (0, 0)),
            pl.BlockSpec((1, S_DIM), lambda i: (0, 0)),
        ],
        out_specs=[
            pl.BlockSpec((BLK, S_DIM), lambda i: (i, 0)),
            pl.BlockSpec((BLK, S_DIM), lambda i: (i, 0)),
        ],
        out_shape=[
            jax.ShapeDtypeStruct((N_PAD, S_DIM), jnp.float32),
            jax.ShapeDtypeStruct((N_PAD, S_DIM), jnp.float32),
        ],
    )(x, wi, bi, wm, bm)


def kernel(x, edge_index, batch, Wi, bi, Wm, bm, Wu, bu, Wo, bo):
    pad = jnp.concatenate(
        [
            jnp.zeros((1, E_PAD - N_EDGES), jnp.int32),
            jnp.full((1, E_PAD - N_EDGES), N_NODES, jnp.int32),
        ],
        axis=0,
    )
    edge3 = jnp.concatenate([edge_index, pad], axis=1).reshape(2, N_CHUNKS, CHUNK)
    batch2 = jnp.concatenate(
        [batch, jnp.full((N_PAD - N_NODES,), N_GRAPHS, jnp.int32)]
    ).reshape(N_PAD // CHUNK, CHUNK)

    x_pad = jnp.concatenate(
        [x, jnp.zeros((N_PAD - N_NODES, F_DIM), jnp.float32)], axis=0
    )
    st0, msg0 = _tc_init(
        x_pad, Wi, bi.reshape(1, S_DIM), Wm[0], bm[0].reshape(1, S_DIM)
    )
    out, _, _ = _sc_mega(
        st0, msg0, edge3, batch2,
        Wm, bm, Wu, bu,
        Wo.reshape(1, S_DIM), jnp.broadcast_to(bo.reshape(1, 1), (1, S_DIM)),
    )
    return out.reshape(-1)


# speedup vs baseline: 1.1989x; 1.0020x over previous
import jax
import jax.numpy as jnp
from jax import lax
from jax.experimental import pallas as pl
from jax.experimental.pallas import tpu as pltpu
from jax.experimental.pallas import tpu_sc as plsc

N_NODES = 10000
N_EDGES = 320000
F_DIM = 128
S_DIM = 16
N_ROUNDS = 4
N_GRAPHS = 64

NC = 2
NS = 16

CHUNK = 128
K_GRP = 8
CH_PER_TILE = 80
N_CHUNKS = NC * NS * CH_PER_TILE
E_PAD = N_CHUNKS * CHUNK
GRPS_PER_TILE = CH_PER_TILE // K_GRP
N_PAD = 10240
ROWS_PER_TILE = N_PAD // NS
RBLK = 128
N_RBLK = ROWS_PER_TILE // RBLK
GS_ROWS = 128

BLK = 1024
N_BLKS = N_PAD // BLK


def _sc_body(st0_hbm, msg0_hbm, edge_hbm, batch_hbm, wm_hbm, bm_hbm, wu_hbm,
             bu_hbm, wo_hbm, bo_hbm, out_hbm, msgscr_hbm, pout_hbm,
             src_v, dst_v, gbuf, zbuf, abuf, pbuf, sbuf, s2buf, mbuf, bidx_v,
             wm_v, bm_v, wu_v, bu_v, wo_v, bo_v, gsbuf, obuf,
             state_sh, agg, gs,
             sem_i, sem_ga, sem_gb, sem_sa, sem_sb, xsem):
    c = lax.axis_index("c")
    s = lax.axis_index("s")
    K = K_GRP
    base = (c * NS + s) * CH_PER_TILE
    rows_t = pl.ds(s * ROWS_PER_TILE, ROWS_PER_TILE)

    pltpu.async_copy(edge_hbm.at[0, pl.ds(base, CH_PER_TILE)], src_v, sem_i)
    pltpu.async_copy(edge_hbm.at[1, pl.ds(base, CH_PER_TILE)], dst_v, sem_i)
    pltpu.sync_copy(st0_hbm.at[rows_t], state_sh.at[rows_t])
    pltpu.sync_copy(wm_hbm, wm_v)
    pltpu.sync_copy(bm_hbm, bm_v)
    pltpu.sync_copy(wu_hbm, wu_v)
    pltpu.sync_copy(bu_hbm, bu_v)
    pltpu.sync_copy(wo_hbm, wo_v)
    pltpu.sync_copy(bo_hbm, bo_v)
    pltpu.sync_copy(batch_hbm.at[pl.ds(N_RBLK * s, N_RBLK)], bidx_v)

    zrow = jnp.zeros((S_DIM,), jnp.float32)

    def _zb(i, _):
        zbuf[i] = zrow
        return 0

    lax.fori_loop(0, ROWS_PER_TILE, _zb, 0)
    pltpu.sync_copy(zbuf, agg.at[rows_t])

    @pl.when(s == 0)
    def _():
        pltpu.sync_copy(zbuf.at[pl.ds(0, GS_ROWS)], gs)

    pltpu.make_async_copy(edge_hbm.at[0, pl.ds(base, CH_PER_TILE)], src_v,
                          sem_i).wait()
    pltpu.make_async_copy(edge_hbm.at[1, pl.ds(base, CH_PER_TILE)], dst_v,
                          sem_i).wait()
    plsc.subcore_barrier()

    sem_g = (sem_ga, sem_gb)
    sem_s = (sem_sa, sem_sb)

    def edge_phase(msg_ref):
        def fire_gathers(g, h):
            for b in range(K):
                pltpu.async_copy(msg_ref.at[src_v.at[g * K + b]],
                                 gbuf.at[h * K + b], sem_g[h])

        def wait_gathers(g, h):
            for b in range(K):
                pltpu.make_async_copy(msg_ref.at[src_v.at[g * K + b]],
                                      gbuf.at[h * K + b], sem_g[h]).wait()

        def fire_scatters(g, h):
            for b in range(K):
                pltpu.async_copy(gbuf.at[h * K + b], agg.at[dst_v.at[g * K + b]],
                                 sem_s[h], add=True)

        def drain_scatters(g, h):
            for b in range(K):
                pltpu.make_async_copy(gbuf.at[h * K + b],
                                      agg.at[dst_v.at[g * K + b]],
                                      sem_s[h]).wait()

        fire_gathers(0, 0)

        def _outer(i, _):
            q = i * 2
            wait_gathers(q, 0)
            fire_scatters(q, 0)

            @pl.when(q > 0)
            def _():
                drain_scatters(q - 1, 1)

            fire_gathers(q + 1, 1)
            wait_gathers(q + 1, 1)
            fire_scatters(q + 1, 1)
            drain_scatters(q, 0)

            @pl.when(q < GRPS_PER_TILE - 2)
            def _():
                fire_gathers(q + 2, 0)

            return 0

        lax.fori_loop(0, GRPS_PER_TILE // 2, _outer, 0)
        drain_scatters(GRPS_PER_TILE - 1, 1)

    def update_phase(r):
        wu_rows = tuple(wu_v[r, k] for k in range(S_DIM))
        bu_row = bu_v[r]
        if r < N_ROUNDS - 1:
            wm_rows = tuple(wm_v[r + 1, k] for k in range(S_DIM))
            bm_row = bm_v[r + 1]
        else:
            wm_rows = ()
            bm_row = bu_row

        def _blk(blk, _):
            rows = pl.ds(s * ROWS_PER_TILE + blk * RBLK, RBLK)
            pltpu.sync_copy(agg.at[rows], abuf)
            pltpu.sync_copy(pout_hbm.at[1 - c, rows], pbuf)
            pltpu.sync_copy(state_sh.at[rows], sbuf)
            pltpu.sync_copy(zbuf.at[pl.ds(0, RBLK)], agg.at[rows])

            def _node(i, w):
                wu_c, bu_c, wm_c, bm_c = w
                av = abuf[i] + pbuf[i]
                acc = bu_c
                for k in range(S_DIM):
                    acc = acc + wu_c[k] * av[k]
                nst = sbuf[i] + jnp.maximum(acc, 0.0)
                s2buf[i] = nst
                if r < N_ROUNDS - 1:
                    macc = bm_c
                    for k in range(S_DIM):
                        macc = macc + wm_c[k] * nst[k]
                    mbuf[i] = jnp.maximum(macc, 0.0)
                return w

            lax.fori_loop(0, RBLK, _node, (wu_rows, bu_row, wm_rows, bm_row))
            pltpu.sync_copy(s2buf, state_sh.at[rows])
            if r < N_ROUNDS - 1:
                pltpu.sync_copy(mbuf, msgscr_hbm.at[c, rows])
            else:
                pltpu.sync_copy(s2buf, gs.at[bidx_v.at[blk]], add=True)
            return 0

        lax.fori_loop(0, N_RBLK, _blk, 0)

    for r in range(N_ROUNDS):
        edge_phase(msg0_hbm if r == 0 else msgscr_hbm.at[c])
        plsc.subcore_barrier()
        pltpu.sync_copy(agg.at[rows_t], pout_hbm.at[c, rows_t])
        pl.semaphore_signal(xsem, 1, core_index=1 - c)
        pl.semaphore_wait(xsem, 1)
        update_phase(r)
        plsc.subcore_barrier()

    @pl.when((c == 0) & (s == 0))
    def _():
        pltpu.sync_copy(gs.at[pl.ds(0, N_GRAPHS)], gsbuf)
        wo_row = wo_v[0]
        bo_s = bo_v[0][0]
        lanes = lax.iota(jnp.int32, 16)

        def _g(rr, _):
            acc = jnp.zeros((16,), jnp.float32)
            for j in range(16):
                val = jnp.sum(gsbuf[rr * 16 + j] * wo_row) + bo_s
                acc = acc + jnp.where(lanes == j, val, 0.0)
            obuf[rr] = acc
            return 0

        lax.fori_loop(0, N_GRAPHS // 16, _g, 0)
        pltpu.sync_copy(obuf, out_hbm)


_sc_mega = pl.kernel(
    _sc_body,
    out_type=(
        jax.ShapeDtypeStruct((N_GRAPHS // 16, 16), jnp.float32),
        jax.ShapeDtypeStruct((NC, N_PAD, S_DIM), jnp.float32),
        jax.ShapeDtypeStruct((NC, N_PAD, S_DIM), jnp.float32),
    ),
    mesh=plsc.VectorSubcoreMesh(core_axis_name="c", subcore_axis_name="s"),
    compiler_params=pltpu.CompilerParams(
        use_tc_tiling_on_sc=False, needs_layout_passes=False
    ),
    scratch_types=[
        pltpu.VMEM((CH_PER_TILE, CHUNK), jnp.int32),
        pltpu.VMEM((CH_PER_TILE, CHUNK), jnp.int32),
        pltpu.VMEM((2 * K_GRP, CHUNK, S_DIM), jnp.float32),
        pltpu.VMEM((ROWS_PER_TILE, S_DIM), jnp.float32),
        pltpu.VMEM((RBLK, S_DIM), jnp.float32),
        pltpu.VMEM((RBLK, S_DIM), jnp.float32),
        pltpu.VMEM((RBLK, S_DIM), jnp.float32),
        pltpu.VMEM((RBLK, S_DIM), jnp.float32),
        pltpu.VMEM((RBLK, S_DIM), jnp.float32),
        pltpu.VMEM((N_RBLK, CHUNK), jnp.int32),
        pltpu.VMEM((N_ROUNDS, S_DIM, S_DIM), jnp.float32),
        pltpu.VMEM((N_ROUNDS, S_DIM), jnp.float32),
        pltpu.VMEM((N_ROUNDS, S_DIM, S_DIM), jnp.float32),
        pltpu.VMEM((N_ROUNDS, S_DIM), jnp.float32),
        pltpu.VMEM((1, S_DIM), jnp.float32),
        pltpu.VMEM((1, S_DIM), jnp.float32),
        pltpu.VMEM((N_GRAPHS, S_DIM), jnp.float32),
        pltpu.VMEM((N_GRAPHS // 16, 16), jnp.float32),
        pltpu.VMEM_SHARED((N_PAD, S_DIM), jnp.float32),
        pltpu.VMEM_SHARED((N_PAD, S_DIM), jnp.float32),
        pltpu.VMEM_SHARED((GS_ROWS, S_DIM), jnp.float32),
        pltpu.SemaphoreType.DMA,
        pltpu.SemaphoreType.DMA,
        pltpu.SemaphoreType.DMA,
        pltpu.SemaphoreType.DMA,
        pltpu.SemaphoreType.DMA,
        pltpu.SemaphoreType.REGULAR,
    ],
)


def _tc_init_body(x_ref, wi_ref, bi_ref, wm_ref, bm_ref, st_ref, msg_ref):
    st = jnp.maximum(
        jnp.dot(x_ref[...], wi_ref[...], preferred_element_type=jnp.float32)
        + bi_ref[...],
        0.0,
    )
    st_ref[...] = st
    msg_ref[...] = jnp.maximum(
        jnp.dot(st, wm_ref[...], preferred_element_type=jnp.float32) + bm_ref[...],
        0.0,
    )


def _tc_init(x, wi, bi, wm, bm):
    return pl.pallas_call(
        _tc_init_body,
        grid=(N_BLKS,),
        in_specs=[
            pl.BlockSpec((BLK, F_DIM), lambda i: (i, 0)),
            pl.BlockSpec((F_DIM, S_DIM), lambda i: (0, 0)),
            pl.BlockSpec((1, S_DIM), lambda i: (0, 0)),
            pl.BlockSpec((S_DIM, S_DIM), lambda i: (0, 0)),
            pl.BlockSpec((1, S_DIM), lambda i: (0, 0)),
        ],
        out_specs=[
            pl.BlockSpec((BLK, S_DIM), lambda i: (i, 0)),
            pl.BlockSpec((BLK, S_DIM), lambda i: (i, 0)),
        ],
        out_shape=[
            jax.ShapeDtypeStruct((N_PAD, S_DIM), jnp.float32),
            jax.ShapeDtypeStruct((N_PAD, S_DIM), jnp.float32),
        ],
    )(x, wi, bi, wm, bm)


def kernel(x, edge_index, batch, Wi, bi, Wm, bm, Wu, bu, Wo, bo):
    pad = jnp.concatenate(
        [
            jnp.zeros((1, E_PAD - N_EDGES), jnp.int32),
            jnp.full((1, E_PAD - N_EDGES), N_NODES, jnp.int32),
        ],
        axis=0,
    )
    edge3 = jnp.concatenate([edge_index, pad], axis=1).reshape(2, N_CHUNKS, CHUNK)
    batch2 = jnp.concatenate(
        [batch, jnp.full((N_PAD - N_NODES,), N_GRAPHS, jnp.int32)]
    ).reshape(N_PAD // CHUNK, CHUNK)

    x_pad = jnp.concatenate(
        [x, jnp.zeros((N_PAD - N_NODES, F_DIM), jnp.float32)], axis=0
    )
    st0, msg0 = _tc_init(
        x_pad, Wi, bi.reshape(1, S_DIM), Wm[0], bm[0].reshape(1, S_DIM)
    )
    out, _, _ = _sc_mega(
        st0, msg0, edge3, batch2,
        Wm, bm, Wu, bu,
        Wo.reshape(1, S_DIM), jnp.broadcast_to(bo.reshape(1, 1), (1, S_DIM)),
    )
    return out.reshape(-1)


# speedup vs baseline: 1.2738x; 1.0624x over previous
import functools

import jax
import jax.numpy as jnp
from jax import lax
from jax.experimental import pallas as pl
from jax.experimental.pallas import tpu as pltpu
from jax.experimental.pallas import tpu_sc as plsc

N_NODES = 10000
N_EDGES = 320000
F_DIM = 128
S_DIM = 16
N_ROUNDS = 4
N_GRAPHS = 64

NC = 2
NS = 16
N_TILES = NC * NS

CHUNK = 128
K_GRP = 8
E_PAD = N_TILES * 80 * CHUNK
N_CHUNKS = E_PAD // CHUNK
CH_PER_TILE = N_CHUNKS // N_TILES
GRPS_PER_TILE = CH_PER_TILE // K_GRP
N_PAD = 10240
ROWS_PER_TILE = N_PAD // NS

BLK = 1000
N_BLKS = N_NODES // BLK


def _sc_edge_body(msg_hbm, edge_hbm, out_hbm, src_v, dst_v, gbuf, zbuf, agg,
                  sem_i, sem_ga, sem_gb, sem_sa, sem_sb):
    c = lax.axis_index("c")
    s = lax.axis_index("s")
    wid = c * NS + s
    base = wid * CH_PER_TILE
    K = K_GRP

    pltpu.async_copy(edge_hbm.at[0, pl.ds(base, CH_PER_TILE)], src_v, sem_i)
    pltpu.async_copy(edge_hbm.at[1, pl.ds(base, CH_PER_TILE)], dst_v, sem_i)

    zrow = jnp.zeros((S_DIM,), jnp.float32)

    def _zb(i, _):
        zbuf[i] = zrow
        return 0

    lax.fori_loop(0, ROWS_PER_TILE, _zb, 0)
    pltpu.sync_copy(zbuf, agg.at[pl.ds(s * ROWS_PER_TILE, ROWS_PER_TILE)])
    pltpu.make_async_copy(edge_hbm.at[0, pl.ds(base, CH_PER_TILE)], src_v,
                          sem_i).wait()
    pltpu.make_async_copy(edge_hbm.at[1, pl.ds(base, CH_PER_TILE)], dst_v,
                          sem_i).wait()

    sem_g = (sem_ga, sem_gb)
    sem_s = (sem_sa, sem_sb)

    def fire_gathers(g, h):
        for b in range(K):
            pltpu.async_copy(msg_hbm.at[src_v.at[g * K + b]], gbuf.at[h * K + b],
                             sem_g[h])

    def wait_gathers(g, h):
        for b in range(K):
            pltpu.make_async_copy(msg_hbm.at[src_v.at[g * K + b]],
                                  gbuf.at[h * K + b], sem_g[h]).wait()

    def fire_scatters(g, h):
        for b in range(K):
            pltpu.async_copy(gbuf.at[h * K + b], agg.at[dst_v.at[g * K + b]],
                             sem_s[h], add=True)

    def drain_scatters(g, h):
        for b in range(K):
            pltpu.make_async_copy(gbuf.at[h * K + b], agg.at[dst_v.at[g * K + b]],
                                  sem_s[h]).wait()

    fire_gathers(0, 0)
    plsc.subcore_barrier()

    def _outer(i, _):
        q = i * 2
        wait_gathers(q, 0)
        fire_scatters(q, 0)

        @pl.when(q > 0)
        def _():
            drain_scatters(q - 1, 1)

        fire_gathers(q + 1, 1)
        wait_gathers(q + 1, 1)
        fire_scatters(q + 1, 1)
        drain_scatters(q, 0)

        @pl.when(q < GRPS_PER_TILE - 2)
        def _():
            fire_gathers(q + 2, 0)

        return 0

    lax.fori_loop(0, GRPS_PER_TILE // 2, _outer, 0)
    drain_scatters(GRPS_PER_TILE - 1, 1)
    plsc.subcore_barrier()

    pltpu.sync_copy(
        agg.at[pl.ds(s * ROWS_PER_TILE, ROWS_PER_TILE)],
        out_hbm.at[c, pl.ds(s * ROWS_PER_TILE, ROWS_PER_TILE)],
    )


_sc_edge = pl.kernel(
    _sc_edge_body,
    out_type=jax.ShapeDtypeStruct((NC, N_PAD, S_DIM), jnp.float32),
    mesh=plsc.VectorSubcoreMesh(core_axis_name="c", subcore_axis_name="s"),
    compiler_params=pltpu.CompilerParams(use_tc_tiling_on_sc=False),
    scratch_types=[
        pltpu.VMEM((CH_PER_TILE, CHUNK), jnp.int32),
        pltpu.VMEM((CH_PER_TILE, CHUNK), jnp.int32),
        pltpu.VMEM((2 * K_GRP, CHUNK, S_DIM), jnp.float32),
        pltpu.VMEM((ROWS_PER_TILE, S_DIM), jnp.float32),
        pltpu.VMEM_SHARED((N_PAD, S_DIM), jnp.float32),
        pltpu.SemaphoreType.DMA,
        pltpu.SemaphoreType.DMA,
        pltpu.SemaphoreType.DMA,
        pltpu.SemaphoreType.DMA,
        pltpu.SemaphoreType.DMA,
    ],
)


def _tc_init_body(x_ref, wi_ref, bi_ref, wm_ref, bm_ref, st_ref, msg_ref):
    st = jnp.maximum(
        jnp.dot(x_ref[...], wi_ref[...], preferred_element_type=jnp.float32)
        + bi_ref[...],
        0.0,
    )
    st_ref[...] = st
    msg_ref[...] = jnp.maximum(
        jnp.dot(st, wm_ref[...], preferred_element_type=jnp.float32) + bm_ref[...],
        0.0,
    )


def _tc_init(x, wi, bi, wm, bm):
    return pl.pallas_call(
        _tc_init_body,
        grid=(N_BLKS,),
        in_specs=[
            pl.BlockSpec((BLK, F_DIM), lambda i: (i, 0)),
            pl.BlockSpec((F_DIM, S_DIM), lambda i: (0, 0)),
            pl.BlockSpec((1, S_DIM), lambda i: (0, 0)),
            pl.BlockSpec((S_DIM, S_DIM), lambda i: (0, 0)),
            pl.BlockSpec((1, S_DIM), lambda i: (0, 0)),
        ],
        out_specs=[
            pl.BlockSpec((BLK, S_DIM), lambda i: (i, 0)),
            pl.BlockSpec((BLK, S_DIM), lambda i: (i, 0)),
        ],
        out_shape=[
            jax.ShapeDtypeStruct((N_NODES, S_DIM), jnp.float32),
            jax.ShapeDtypeStruct((N_NODES, S_DIM), jnp.float32),
        ],
    )(x, wi, bi, wm, bm)


def _tc_upd_body(st_ref, p_ref, wu_ref, bu_ref, wm_ref, bm_ref, st_out, msg_out):
    a = p_ref[0] + p_ref[1]
    st = st_ref[...] + jnp.maximum(
        jnp.dot(a, wu_ref[...], preferred_element_type=jnp.float32) + bu_ref[...],
        0.0,
    )
    st_out[...] = st
    msg_out[...] = jnp.maximum(
        jnp.dot(st, wm_ref[...], preferred_element_type=jnp.float32) + bm_ref[...],
        0.0,
    )


def _tc_upd(st, parts, wu, bu, wm, bm):
    return pl.pallas_call(
        _tc_upd_body,
        grid=(N_BLKS,),
        in_specs=[
            pl.BlockSpec((BLK, S_DIM), lambda i: (i, 0)),
            pl.BlockSpec((NC, BLK, S_DIM), lambda i: (0, i, 0)),
            pl.BlockSpec((S_DIM, S_DIM), lambda i: (0, 0)),
            pl.BlockSpec((1, S_DIM), lambda i: (0, 0)),
            pl.BlockSpec((S_DIM, S_DIM), lambda i: (0, 0)),
            pl.BlockSpec((1, S_DIM), lambda i: (0, 0)),
        ],
        out_specs=[
            pl.BlockSpec((BLK, S_DIM), lambda i: (i, 0)),
            pl.BlockSpec((BLK, S_DIM), lambda i: (i, 0)),
        ],
        out_shape=[
            jax.ShapeDtypeStruct((N_NODES, S_DIM), jnp.float32),
            jax.ShapeDtypeStruct((N_NODES, S_DIM), jnp.float32),
        ],
    )(st, parts, wu, bu, wm, bm)


def _tc_fin_body(st_ref, p_ref, wu_ref, bu_ref, b_ref, wo_ref, bo_ref, out_ref, gs_ref):
    i = pl.program_id(0)

    @pl.when(i == 0)
    def _():
        gs_ref[...] = jnp.zeros_like(gs_ref)

    a = p_ref[0] + p_ref[1]
    st = st_ref[...] + jnp.maximum(
        jnp.dot(a, wu_ref[...], preferred_element_type=jnp.float32) + bu_ref[...],
        0.0,
    )
    b = b_ref[0, 0, :]
    onehot = (
        lax.broadcasted_iota(jnp.int32, (N_GRAPHS, BLK), 0) == b[None, :]
    ).astype(jnp.float32)
    gs_ref[...] += jnp.dot(onehot, st, preferred_element_type=jnp.float32)

    @pl.when(i == pl.num_programs(0) - 1)
    def _():
        out_ref[...] = (
            jnp.dot(gs_ref[...], wo_ref[...], preferred_element_type=jnp.float32)
            + bo_ref[...]
        )


def _tc_fin(st, parts, wu, bu, batch3, wo, bo):
    return pl.pallas_call(
        _tc_fin_body,
        grid=(N_BLKS,),
        in_specs=[
            pl.BlockSpec((BLK, S_DIM), lambda i: (i, 0)),
            pl.BlockSpec((NC, BLK, S_DIM), lambda i: (0, i, 0)),
            pl.BlockSpec((S_DIM, S_DIM), lambda i: (0, 0)),
            pl.BlockSpec((1, S_DIM), lambda i: (0, 0)),
            pl.BlockSpec((1, 1, BLK), lambda i: (i, 0, 0)),
            pl.BlockSpec((S_DIM, 1), lambda i: (0, 0)),
            pl.BlockSpec((1, 1), lambda i: (0, 0)),
        ],
        out_specs=pl.BlockSpec((N_GRAPHS, 1), lambda i: (0, 0)),
        out_shape=jax.ShapeDtypeStruct((N_GRAPHS, 1), jnp.float32),
        scratch_shapes=[pltpu.VMEM((N_GRAPHS, S_DIM), jnp.float32)],
    )(st, parts, wu, bu, batch3, wo, bo)


def kernel(x, edge_index, batch, Wi, bi, Wm, bm, Wu, bu, Wo, bo):
    pad = jnp.concatenate(
        [
            jnp.zeros((1, E_PAD - N_EDGES), jnp.int32),
            jnp.full((1, E_PAD - N_EDGES), N_NODES, jnp.int32),
        ],
        axis=0,
    )
    edge3 = jnp.concatenate([edge_index, pad], axis=1).reshape(2, N_CHUNKS, CHUNK)
    batch3 = batch.reshape(N_BLKS, 1, BLK)
    bi2 = bi.reshape(1, S_DIM)
    bo2 = bo.reshape(1, 1)

    st, msg = _tc_init(x, Wi, bi2, Wm[0], bm[0].reshape(1, S_DIM))
    for r in range(N_ROUNDS):
        parts = _sc_edge(msg, edge3)
        if r < N_ROUNDS - 1:
            st, msg = _tc_upd(
                st, parts,
                Wu[r], bu[r].reshape(1, S_DIM),
                Wm[r + 1], bm[r + 1].reshape(1, S_DIM),
            )
        else:
            out = _tc_fin(st, parts, Wu[r], bu[r].reshape(1, S_DIM), batch3, Wo, bo2)
    return out.reshape(-1)
